# baseline (speedup 1.0000x reference)
import functools

import jax
import jax.numpy as jnp
from jax.experimental import pallas as pl
from jax.experimental.pallas import tpu as pltpu

B = 32
F = 64
BMH = 256
NS = 2 * BMH
GH = 8


def _score_copy_body(x_ref, w1a_ref, w1btp_ref, b1_ref, w2p_ref, thr_ref,
                     mem_ref, out_mem_ref, patch_ref, mask_ref, groups_ref,
                     m_s, s_s, idx_s):
    i = pl.program_id(0)
    nblk = pl.num_programs(0)

    @pl.when(i == 0)
    def _init():
        m_s[...] = jnp.full((B, 1), -jnp.inf, jnp.float32)
        s_s[...] = jnp.zeros((B, 1), jnp.float32)
        idx_s[...] = jnp.zeros((B, 1), jnp.int32)

    x = x_ref[...]
    in_proj = jnp.dot(x, w1a_ref[...],
                      preferred_element_type=jnp.float32) + b1_ref[...]
    packed = mem_ref[...]
    out_mem_ref[...] = packed
    projT = jax.lax.dot_general(
        w1btp_ref[...], packed, (((1,), (1,)), ((), ())),
        preferred_element_type=jnp.float32)

    in_projT = in_proj.T
    w2p = w2p_ref[...]
    se_rows, so_rows = [], []
    for b in range(B):
        ipb = in_projT[:, b:b + 1]
        argb = projT + jnp.concatenate([ipb, ipb], axis=0)
        sb = jnp.dot(w2p, jnp.tanh(argb),
                     preferred_element_type=jnp.float32)
        se_rows.append(sb[0:1, :])
        so_rows.append(sb[1:2, :])
    scores_e = jnp.concatenate(se_rows, axis=0)
    scores_o = jnp.concatenate(so_rows, axis=0)

    max_e = jnp.max(scores_e, axis=1, keepdims=True)
    max_o = jnp.max(scores_o, axis=1, keepdims=True)
    arg_e = jnp.argmax(scores_e, axis=1).astype(jnp.int32).reshape(B, 1)
    arg_o = jnp.argmax(scores_o, axis=1).astype(jnp.int32).reshape(B, 1)
    use_e = (max_e > max_o) | ((max_e == max_o) & (arg_e <= arg_o))
    blk_max = jnp.maximum(max_e, max_o)
    blk_arg = jnp.where(use_e, 2 * arg_e, 2 * arg_o + 1) + i * NS
    m_old = m_s[...]
    m_new = jnp.maximum(m_old, blk_max)
    s_new = (s_s[...] * jnp.exp(m_old - m_new)
             + jnp.sum(jnp.exp(scores_e - m_new), axis=1, keepdims=True)
             + jnp.sum(jnp.exp(scores_o - m_new), axis=1, keepdims=True))
    idx_s[...] = jnp.where(blk_max > m_old, blk_arg, idx_s[...])
    m_s[...] = m_new
    s_s[...] = s_new

    @pl.when(i == nblk - 1)
    def _finalize():
        best_w = 1.0 / s_s[...]
        do_write = best_w > thr_ref[...]
        slot = idx_s[...]
        eq = slot == slot.reshape(1, B)
        ii = jax.lax.broadcasted_iota(jnp.int32, (B, B), 0)
        jj = jax.lax.broadcasted_iota(jnp.int32, (B, B), 1)
        conflict = jnp.any(eq & (jj > ii) & do_write.reshape(1, B),
                           axis=1, keepdims=True)
        final_write = do_write & jnp.logical_not(conflict)
        group = slot // (2 * GH)
        qrow = (slot % (2 * GH)) // 2
        par = slot % 2
        q8 = jax.lax.broadcasted_iota(jnp.int32, (1, GH, 1), 1)
        same = (final_write.reshape(1, 1, B)
                & (group.reshape(1, 1, B) == group.reshape(B, 1, 1))
                & (qrow.reshape(1, 1, B) == q8))
        match_e = jnp.where(same & (par.reshape(1, 1, B) == 0),
                            1.0, 0.0).reshape(B * GH, B)
        match_o = jnp.where(same & (par.reshape(1, 1, B) == 1),
                            1.0, 0.0).reshape(B * GH, B)
        zero = jnp.zeros((B, F), jnp.float32)
        xl = jnp.concatenate([x, zero], axis=1)
        xr = jnp.concatenate([zero, x], axis=1)
        patch_ref[...] = (
            jnp.dot(match_e, xl, preferred_element_type=jnp.float32)
            + jnp.dot(match_o, xr,
                      preferred_element_type=jnp.float32))
        rme = jnp.sum(match_e, axis=1, keepdims=True)
        rmo = jnp.sum(match_o, axis=1, keepdims=True)
        lane = jax.lax.broadcasted_iota(jnp.int32, (B * GH, 128), 1)
        halff = (lane // F).astype(jnp.float32)
        mask_ref[...] = rme * (1.0 - halff) + rmo * halff
        groups_ref[...] = group.reshape(1, B)


def _scatter_body(groups_ref, patch_ref, mask_ref, cur_ref, out_ref):
    out_ref[...] = jnp.where(mask_ref[...] != 0.0,
                             patch_ref[...], cur_ref[...])


@functools.partial(jax.jit, static_argnames=())
def kernel(input_data, memory, W1, b1, W2, b2, threshold):
    del b2
    M = memory.shape[0]
    MH = M // 2
    nblk = MH // BMH

    w1a = W1[:F, :]
    w1bt = W1[F:, :].T
    zf = jnp.zeros((F, F), jnp.float32)
    w1btp = jnp.concatenate([
        jnp.concatenate([w1bt, zf], axis=1),
        jnp.concatenate([zf, w1bt], axis=1)], axis=0)
    w2row = W2.reshape(1, F)
    z1 = jnp.zeros((1, F), jnp.float32)
    w2p = jnp.concatenate([
        jnp.concatenate([w2row, z1], axis=1),
        jnp.concatenate([z1, w2row], axis=1),
        jnp.zeros((6, 2 * F), jnp.float32)], axis=0)
    b1r = b1.reshape(1, F)
    thr = threshold.reshape(1, 1)
    memp = memory.reshape(MH, 2 * F)

    out_mem, patch, mask, groups = pl.pallas_call(
        _score_copy_body,
        grid=(nblk,),
        in_specs=[
            pl.BlockSpec((B, F), lambda i: (0, 0)),
            pl.BlockSpec((F, F), lambda i: (0, 0)),
            pl.BlockSpec((2 * F, 2 * F), lambda i: (0, 0)),
            pl.BlockSpec((1, F), lambda i: (0, 0)),
            pl.BlockSpec((8, 2 * F), lambda i: (0, 0)),
            pl.BlockSpec((1, 1), lambda i: (0, 0)),
            pl.BlockSpec((BMH, 2 * F), lambda i: (i, 0)),
        ],
        out_specs=[
            pl.BlockSpec((BMH, 2 * F), lambda i: (i, 0)),
            pl.BlockSpec((B * GH, 2 * F), lambda i: (0, 0)),
            pl.BlockSpec((B * GH, 2 * F), lambda i: (0, 0)),
            pl.BlockSpec((1, B), lambda i: (0, 0)),
        ],
        out_shape=[
            jax.ShapeDtypeStruct((MH, 2 * F), jnp.float32),
            jax.ShapeDtypeStruct((B * GH, 2 * F), jnp.float32),
            jax.ShapeDtypeStruct((B * GH, 2 * F), jnp.float32),
            jax.ShapeDtypeStruct((1, B), jnp.int32),
        ],
        scratch_shapes=[
            pltpu.VMEM((B, 1), jnp.float32),
            pltpu.VMEM((B, 1), jnp.float32),
            pltpu.VMEM((B, 1), jnp.int32),
        ],
    )(input_data, w1a, w1btp, b1r, w2p, thr, memp)

    groups1d = groups.reshape(B)

    grid_spec = pltpu.PrefetchScalarGridSpec(
        num_scalar_prefetch=1,
        grid=(B,),
        in_specs=[
            pl.BlockSpec((GH, 2 * F), lambda i, g: (i, 0)),
            pl.BlockSpec((GH, 2 * F), lambda i, g: (i, 0)),
            pl.BlockSpec((GH, 2 * F), lambda i, g: (g[i], 0)),
        ],
        out_specs=pl.BlockSpec((GH, 2 * F), lambda i, g: (g[i], 0)),
    )
    updated = pl.pallas_call(
        _scatter_body,
        grid_spec=grid_spec,
        out_shape=jax.ShapeDtypeStruct((MH, 2 * F), jnp.float32),
        input_output_aliases={3: 0},
    )(groups1d, patch, mask, out_mem)
    return updated.reshape(M, F)


# speedup vs baseline: 1.2480x; 1.2480x over previous
import functools

import jax
import jax.numpy as jnp
from jax.experimental import pallas as pl
from jax.experimental.pallas import tpu as pltpu

B = 32
F = 64
BM = 512
G = 8


def _score_copy_body(x_ref, w1a_ref, w1bt_ref, b1_ref, w2blk_ref, thr_ref,
                     mem_ref, out_mem_ref, patch_ref, mask_ref, groups_ref,
                     m_s, s_s, idx_s):
    i = pl.program_id(0)
    nblk = pl.num_programs(0)

    @pl.when(i == 0)
    def _init():
        m_s[...] = jnp.full((B, 1), -jnp.inf, jnp.float32)
        s_s[...] = jnp.zeros((B, 1), jnp.float32)
        idx_s[...] = jnp.zeros((B, 1), jnp.int32)

    x = x_ref[...]
    in_proj = jnp.dot(x, w1a_ref[...],
                      preferred_element_type=jnp.float32) + b1_ref[...]
    memb = mem_ref[...]
    out_mem_ref[...] = memb
    mem_projT = jax.lax.dot_general(
        w1bt_ref[...], memb, (((1,), (1,)), ((), ())),
        preferred_element_type=jnp.float32)

    w2blk = w2blk_ref[...]
    tiles = []
    for g in range(B // G):
        ipg = in_proj[g * G:(g + 1) * G, :]
        slab = jnp.tanh(mem_projT[None, :, :]
                        + ipg[:, :, None]).reshape(G * F, BM)
        tiles.append(jnp.dot(w2blk, slab,
                             preferred_element_type=jnp.float32))
    scores = jnp.concatenate(tiles, axis=0)

    blk_max = jnp.max(scores, axis=1, keepdims=True)
    blk_arg = (jnp.argmax(scores, axis=1).astype(jnp.int32).reshape(B, 1)
               + i * BM)
    m_old = m_s[...]
    m_new = jnp.maximum(m_old, blk_max)
    s_new = (s_s[...] * jnp.exp(m_old - m_new)
             + jnp.sum(jnp.exp(scores - m_new), axis=1, keepdims=True))
    idx_s[...] = jnp.where(blk_max > m_old, blk_arg, idx_s[...])
    m_s[...] = m_new
    s_s[...] = s_new

    @pl.when(i == nblk - 1)
    def _finalize():
        best_w = 1.0 / s_s[...]
        do_write = best_w > thr_ref[...]
        slot = idx_s[...]
        eq = slot == slot.reshape(1, B)
        ii = jax.lax.broadcasted_iota(jnp.int32, (B, B), 0)
        jj = jax.lax.broadcasted_iota(jnp.int32, (B, B), 1)
        conflict = jnp.any(eq & (jj > ii) & do_write.reshape(1, B),
                           axis=1, keepdims=True)
        final_write = do_write & jnp.logical_not(conflict)
        group = slot // G
        row = slot % G
        r8 = jax.lax.broadcasted_iota(jnp.int32, (1, G, 1), 1)
        match3 = (final_write.reshape(1, 1, B)
                  & (group.reshape(1, 1, B) == group.reshape(B, 1, 1))
                  & (row.reshape(1, 1, B) == r8))
        match2 = jnp.where(match3, 1.0, 0.0).reshape(B * G, B)
        mask_ref[...] = jnp.sum(match2, axis=1, keepdims=True)
        patch_ref[...] = jnp.dot(match2, x,
                                 preferred_element_type=jnp.float32)
        groups_ref[...] = group.reshape(1, B)


def _scatter_body(groups_ref, patch_ref, mask_ref, cur_ref, out_ref):
    out_ref[...] = jnp.where(mask_ref[...] != 0.0,
                             patch_ref[...], cur_ref[...])


@functools.partial(jax.jit, static_argnames=())
def kernel(input_data, memory, W1, b1, W2, b2, threshold):
    del b2
    M = memory.shape[0]
    nblk = M // BM

    w1a = W1[:F, :]
    w1bt = W1[F:, :].T
    b1r = b1.reshape(1, F)
    thr = threshold.reshape(1, 1)
    w2row = W2.reshape(1, F)
    w2blk = jnp.zeros((G, G * F), jnp.float32)
    for g in range(G):
        w2blk = w2blk.at[g:g + 1, g * F:(g + 1) * F].set(w2row)

    out_mem, patch, mask, groups = pl.pallas_call(
        _score_copy_body,
        grid=(nblk,),
        in_specs=[
            pl.BlockSpec((B, F), lambda i: (0, 0)),
            pl.BlockSpec((F, F), lambda i: (0, 0)),
            pl.BlockSpec((F, F), lambda i: (0, 0)),
            pl.BlockSpec((1, F), lambda i: (0, 0)),
            pl.BlockSpec((G, G * F), lambda i: (0, 0)),
            pl.BlockSpec((1, 1), lambda i: (0, 0)),
            pl.BlockSpec((BM, F), lambda i: (i, 0)),
        ],
        out_specs=[
            pl.BlockSpec((BM, F), lambda i: (i, 0)),
            pl.BlockSpec((B * G, F), lambda i: (0, 0)),
            pl.BlockSpec((B * G, 1), lambda i: (0, 0)),
            pl.BlockSpec((1, B), lambda i: (0, 0)),
        ],
        out_shape=[
            jax.ShapeDtypeStruct((M, F), jnp.float32),
            jax.ShapeDtypeStruct((B * G, F), jnp.float32),
            jax.ShapeDtypeStruct((B * G, 1), jnp.float32),
            jax.ShapeDtypeStruct((1, B), jnp.int32),
        ],
        scratch_shapes=[
            pltpu.VMEM((B, 1), jnp.float32),
            pltpu.VMEM((B, 1), jnp.float32),
            pltpu.VMEM((B, 1), jnp.int32),
        ],
    )(input_data, w1a, w1bt, b1r, w2blk, thr, memory)

    groups1d = groups.reshape(B)

    grid_spec = pltpu.PrefetchScalarGridSpec(
        num_scalar_prefetch=1,
        grid=(B,),
        in_specs=[
            pl.BlockSpec((G, F), lambda i, g: (i, 0)),
            pl.BlockSpec((G, 1), lambda i, g: (i, 0)),
            pl.BlockSpec((G, F), lambda i, g: (g[i], 0)),
        ],
        out_specs=pl.BlockSpec((G, F), lambda i, g: (g[i], 0)),
    )
    updated = pl.pallas_call(
        _scatter_body,
        grid_spec=grid_spec,
        out_shape=jax.ShapeDtypeStruct((M, F), jnp.float32),
        input_output_aliases={3: 0},
    )(groups1d, patch, mask, out_mem)
    return updated


# speedup vs baseline: 1.3636x; 1.0926x over previous
import functools

import jax
import jax.numpy as jnp
from jax.experimental import pallas as pl
from jax.experimental.pallas import tpu as pltpu

B = 32
F = 64
BM = 512
G = 8


def _score_copy_body(x_ref, w1a_ref, w1bt_ref, b1_ref, w2blk_ref, thr_ref,
                     mem_ref, out_mem_ref, patch_ref, mask_ref, groups_ref,
                     m_s, s_s, idx_s):
    i = pl.program_id(0)
    nblk = pl.num_programs(0)

    @pl.when(i == 0)
    def _init():
        m_s[...] = jnp.full((B, 128), -jnp.inf, jnp.float32)
        s_s[...] = jnp.zeros((B, 128), jnp.float32)
        idx_s[...] = jnp.zeros((B, 128), jnp.int32)

    x = x_ref[...]
    in_proj = jnp.dot(x, w1a_ref[...],
                      preferred_element_type=jnp.float32) + b1_ref[...]
    memb = mem_ref[...]
    out_mem_ref[...] = memb
    mem_projT = jax.lax.dot_general(
        w1bt_ref[...], memb, (((1,), (1,)), ((), ())),
        preferred_element_type=jnp.float32)

    w2blk = w2blk_ref[...]
    tiles = []
    for g in range(B // G):
        ipg = in_proj[g * G:(g + 1) * G, :]
        slab = jnp.tanh(mem_projT[None, :, :]
                        + ipg[:, :, None]).reshape(G * F, BM)
        tiles.append(jnp.dot(w2blk, slab,
                             preferred_element_type=jnp.float32))
    scores = jnp.concatenate(tiles, axis=0)

    lane_idx = jax.lax.broadcasted_iota(jnp.int32, (B, 128), 1)
    for t in range(BM // 128):
        tile = scores[:, t * 128:(t + 1) * 128]
        m_old = m_s[...]
        m_new = jnp.maximum(m_old, tile)
        s_s[...] = s_s[...] * jnp.exp(m_old - m_new) + jnp.exp(tile - m_new)
        idx_s[...] = jnp.where(tile > m_old,
                               i * BM + t * 128 + lane_idx, idx_s[...])
        m_s[...] = m_new

    @pl.when(i == nblk - 1)
    def _finalize():
        m_lane = m_s[...]
        m_g = jnp.max(m_lane, axis=1, keepdims=True)
        s = jnp.sum(s_s[...] * jnp.exp(m_lane - m_g),
                    axis=1, keepdims=True)
        cand = jnp.where(m_lane == m_g, idx_s[...], jnp.int32(2 ** 30))
        slot = jnp.min(cand, axis=1, keepdims=True)
        best_w = 1.0 / s
        do_write = best_w > thr_ref[...]
        eq = slot == slot.reshape(1, B)
        ii = jax.lax.broadcasted_iota(jnp.int32, (B, B), 0)
        jj = jax.lax.broadcasted_iota(jnp.int32, (B, B), 1)
        conflict = jnp.any(eq & (jj > ii) & do_write.reshape(1, B),
                           axis=1, keepdims=True)
        final_write = do_write & jnp.logical_not(conflict)
        group = slot // G
        row = slot % G
        r8 = jax.lax.broadcasted_iota(jnp.int32, (1, G, 1), 1)
        match3 = (final_write.reshape(1, 1, B)
                  & (group.reshape(1, 1, B) == group.reshape(B, 1, 1))
                  & (row.reshape(1, 1, B) == r8))
        match2 = jnp.where(match3, 1.0, 0.0).reshape(B * G, B)
        mask_ref[...] = jnp.sum(match2, axis=1, keepdims=True)
        patch_ref[...] = jnp.dot(match2, x,
                                 preferred_element_type=jnp.float32)
        groups_ref[...] = group.reshape(1, B)


def _scatter_body(groups_ref, patch_ref, mask_ref, cur_ref, out_ref):
    out_ref[...] = jnp.where(mask_ref[...] != 0.0,
                             patch_ref[...], cur_ref[...])


@functools.partial(jax.jit, static_argnames=())
def kernel(input_data, memory, W1, b1, W2, b2, threshold):
    del b2
    M = memory.shape[0]
    nblk = M // BM

    w1a = W1[:F, :]
    w1bt = W1[F:, :].T
    b1r = b1.reshape(1, F)
    thr = threshold.reshape(1, 1)
    w2row = W2.reshape(1, F)
    w2blk = jnp.zeros((G, G * F), jnp.float32)
    for g in range(G):
        w2blk = w2blk.at[g:g + 1, g * F:(g + 1) * F].set(w2row)

    out_mem, patch, mask, groups = pl.pallas_call(
        _score_copy_body,
        grid=(nblk,),
        in_specs=[
            pl.BlockSpec((B, F), lambda i: (0, 0)),
            pl.BlockSpec((F, F), lambda i: (0, 0)),
            pl.BlockSpec((F, F), lambda i: (0, 0)),
            pl.BlockSpec((1, F), lambda i: (0, 0)),
            pl.BlockSpec((G, G * F), lambda i: (0, 0)),
            pl.BlockSpec((1, 1), lambda i: (0, 0)),
            pl.BlockSpec((BM, F), lambda i: (i, 0)),
        ],
        out_specs=[
            pl.BlockSpec((BM, F), lambda i: (i, 0)),
            pl.BlockSpec((B * G, F), lambda i: (0, 0)),
            pl.BlockSpec((B * G, 1), lambda i: (0, 0)),
            pl.BlockSpec((1, B), lambda i: (0, 0)),
        ],
        out_shape=[
            jax.ShapeDtypeStruct((M, F), jnp.float32),
            jax.ShapeDtypeStruct((B * G, F), jnp.float32),
            jax.ShapeDtypeStruct((B * G, 1), jnp.float32),
            jax.ShapeDtypeStruct((1, B), jnp.int32),
        ],
        scratch_shapes=[
            pltpu.VMEM((B, 128), jnp.float32),
            pltpu.VMEM((B, 128), jnp.float32),
            pltpu.VMEM((B, 128), jnp.int32),
        ],
    )(input_data, w1a, w1bt, b1r, w2blk, thr, memory)

    groups1d = groups.reshape(B)

    grid_spec = pltpu.PrefetchScalarGridSpec(
        num_scalar_prefetch=1,
        grid=(B,),
        in_specs=[
            pl.BlockSpec((G, F), lambda i, g: (i, 0)),
            pl.BlockSpec((G, 1), lambda i, g: (i, 0)),
            pl.BlockSpec((G, F), lambda i, g: (g[i], 0)),
        ],
        out_specs=pl.BlockSpec((G, F), lambda i, g: (g[i], 0)),
    )
    updated = pl.pallas_call(
        _scatter_body,
        grid_spec=grid_spec,
        out_shape=jax.ShapeDtypeStruct((M, F), jnp.float32),
        input_output_aliases={3: 0},
    )(groups1d, patch, mask, out_mem)
    return updated


# speedup vs baseline: 1.3685x; 1.0035x over previous
import functools

import jax
import jax.numpy as jnp
from jax.experimental import pallas as pl
from jax.experimental.pallas import tpu as pltpu

B = 32
F = 64
BM = 512
G = 8


def _score_copy_body(x_ref, w1a_ref, w1bt_ref, b1_ref, w2blk_ref, thr_ref,
                     mem_ref, out_mem_ref, patch_ref, mask_ref, groups_ref,
                     m_s, s_s, idx_s):
    i = pl.program_id(0)
    nblk = pl.num_programs(0)

    @pl.when(i == 0)
    def _init():
        m_s[...] = jnp.full((B, 128), -jnp.inf, jnp.float32)
        s_s[...] = jnp.zeros((B, 128), jnp.float32)
        idx_s[...] = jnp.zeros((B, 128), jnp.int32)

    x = x_ref[...]
    in_proj = jnp.dot(x, w1a_ref[...],
                      preferred_element_type=jnp.float32) + b1_ref[...]
    memb = mem_ref[...]
    out_mem_ref[...] = memb
    mem_projT = jax.lax.dot_general(
        w1bt_ref[...], memb, (((1,), (1,)), ((), ())),
        preferred_element_type=jnp.float32)

    w2blk = w2blk_ref[...]
    tiles = []
    for g in range(B // G):
        ipg = in_proj[g * G:(g + 1) * G, :]
        slab = jnp.tanh(mem_projT[None, :, :]
                        + ipg[:, :, None]).reshape(G * F, BM)
        tiles.append(jnp.dot(w2blk, slab,
                             preferred_element_type=jnp.float32))
    scores = jnp.concatenate(tiles, axis=0)

    lane_idx = jax.lax.broadcasted_iota(jnp.int32, (B, 128), 1)
    ntile = BM // 128
    tiles_s = [scores[:, t * 128:(t + 1) * 128] for t in range(ntile)]
    m_old = m_s[...]
    m_new = m_old
    for t in range(ntile):
        m_new = jnp.maximum(m_new, tiles_s[t])
    s_acc = jnp.exp(tiles_s[0] - m_new)
    for t in range(1, ntile):
        s_acc = s_acc + jnp.exp(tiles_s[t] - m_new)
    s_s[...] = s_s[...] * jnp.exp(m_old - m_new) + s_acc
    improved = m_new > m_old
    idx = idx_s[...]
    for t in reversed(range(ntile)):
        idx = jnp.where(improved & (tiles_s[t] == m_new),
                        i * BM + t * 128 + lane_idx, idx)
    idx_s[...] = idx
    m_s[...] = m_new

    @pl.when(i == nblk - 1)
    def _finalize():
        m_lane = m_s[...]
        m_g = jnp.max(m_lane, axis=1, keepdims=True)
        s = jnp.sum(s_s[...] * jnp.exp(m_lane - m_g),
                    axis=1, keepdims=True)
        cand = jnp.where(m_lane == m_g, idx_s[...], jnp.int32(2 ** 30))
        slot = jnp.min(cand, axis=1, keepdims=True)
        best_w = 1.0 / s
        do_write = best_w > thr_ref[...]
        eq = slot == slot.reshape(1, B)
        ii = jax.lax.broadcasted_iota(jnp.int32, (B, B), 0)
        jj = jax.lax.broadcasted_iota(jnp.int32, (B, B), 1)
        conflict = jnp.any(eq & (jj > ii) & do_write.reshape(1, B),
                           axis=1, keepdims=True)
        final_write = do_write & jnp.logical_not(conflict)
        group = slot // G
        row = slot % G
        r8 = jax.lax.broadcasted_iota(jnp.int32, (1, G, 1), 1)
        match3 = (final_write.reshape(1, 1, B)
                  & (group.reshape(1, 1, B) == group.reshape(B, 1, 1))
                  & (row.reshape(1, 1, B) == r8))
        match2 = jnp.where(match3, 1.0, 0.0).reshape(B * G, B)
        mask_ref[...] = jnp.sum(match2, axis=1, keepdims=True)
        patch_ref[...] = jnp.dot(match2, x,
                                 preferred_element_type=jnp.float32)
        groups_ref[...] = group.reshape(1, B)


def _scatter_body(groups_ref, patch_ref, mask_ref, cur_ref, out_ref):
    out_ref[...] = jnp.where(mask_ref[...] != 0.0,
                             patch_ref[...], cur_ref[...])


@functools.partial(jax.jit, static_argnames=())
def kernel(input_data, memory, W1, b1, W2, b2, threshold):
    del b2
    M = memory.shape[0]
    nblk = M // BM

    w1a = W1[:F, :]
    w1bt = W1[F:, :].T
    b1r = b1.reshape(1, F)
    thr = threshold.reshape(1, 1)
    w2row = W2.reshape(1, F)
    w2blk = jnp.zeros((G, G * F), jnp.float32)
    for g in range(G):
        w2blk = w2blk.at[g:g + 1, g * F:(g + 1) * F].set(w2row)

    out_mem, patch, mask, groups = pl.pallas_call(
        _score_copy_body,
        grid=(nblk,),
        in_specs=[
            pl.BlockSpec((B, F), lambda i: (0, 0)),
            pl.BlockSpec((F, F), lambda i: (0, 0)),
            pl.BlockSpec((F, F), lambda i: (0, 0)),
            pl.BlockSpec((1, F), lambda i: (0, 0)),
            pl.BlockSpec((G, G * F), lambda i: (0, 0)),
            pl.BlockSpec((1, 1), lambda i: (0, 0)),
            pl.BlockSpec((BM, F), lambda i: (i, 0)),
        ],
        out_specs=[
            pl.BlockSpec((BM, F), lambda i: (i, 0)),
            pl.BlockSpec((B * G, F), lambda i: (0, 0)),
            pl.BlockSpec((B * G, 1), lambda i: (0, 0)),
            pl.BlockSpec((1, B), lambda i: (0, 0)),
        ],
        out_shape=[
            jax.ShapeDtypeStruct((M, F), jnp.float32),
            jax.ShapeDtypeStruct((B * G, F), jnp.float32),
            jax.ShapeDtypeStruct((B * G, 1), jnp.float32),
            jax.ShapeDtypeStruct((1, B), jnp.int32),
        ],
        scratch_shapes=[
            pltpu.VMEM((B, 128), jnp.float32),
            pltpu.VMEM((B, 128), jnp.float32),
            pltpu.VMEM((B, 128), jnp.int32),
        ],
    )(input_data, w1a, w1bt, b1r, w2blk, thr, memory)

    groups1d = groups.reshape(B)

    grid_spec = pltpu.PrefetchScalarGridSpec(
        num_scalar_prefetch=1,
        grid=(B,),
        in_specs=[
            pl.BlockSpec((G, F), lambda i, g: (i, 0)),
            pl.BlockSpec((G, 1), lambda i, g: (i, 0)),
            pl.BlockSpec((G, F), lambda i, g: (g[i], 0)),
        ],
        out_specs=pl.BlockSpec((G, F), lambda i, g: (g[i], 0)),
    )
    updated = pl.pallas_call(
        _scatter_body,
        grid_spec=grid_spec,
        out_shape=jax.ShapeDtypeStruct((M, F), jnp.float32),
        input_output_aliases={3: 0},
    )(groups1d, patch, mask, out_mem)
    return updated


# speedup vs baseline: 1.6262x; 1.1883x over previous
import functools

import jax
import jax.numpy as jnp
from jax.experimental import pallas as pl
from jax.experimental.pallas import tpu as pltpu

B = 32
F = 64
BM = 1024
G = 8


def _score_copy_body(x_ref, w1a_ref, w1bt_ref, b1_ref, w2blk_ref, thr_ref,
                     mem_ref, out_mem_ref, patch_ref, mask_ref, groups_ref,
                     m_s, s_s, idx_s):
    i = pl.program_id(0)
    nblk = pl.num_programs(0)

    @pl.when(i == 0)
    def _init():
        m_s[...] = jnp.full((B, 128), -jnp.inf, jnp.float32)
        s_s[...] = jnp.zeros((B, 128), jnp.float32)
        idx_s[...] = jnp.zeros((B, 128), jnp.int32)

    x = x_ref[...]
    in_proj = jnp.dot(x, w1a_ref[...],
                      preferred_element_type=jnp.float32) + b1_ref[...]
    memb = mem_ref[...]
    out_mem_ref[...] = memb
    mem_projT = jax.lax.dot_general(
        w1bt_ref[...], memb, (((1,), (1,)), ((), ())),
        preferred_element_type=jnp.float32)

    w2blk = w2blk_ref[...]
    tiles = []
    for g in range(B // G):
        ipg = in_proj[g * G:(g + 1) * G, :]
        slab = jnp.tanh(mem_projT[None, :, :]
                        + ipg[:, :, None]).reshape(G * F, BM)
        tiles.append(jnp.dot(w2blk, slab,
                             preferred_element_type=jnp.float32))
    scores = jnp.concatenate(tiles, axis=0)

    lane_idx = jax.lax.broadcasted_iota(jnp.int32, (B, 128), 1)
    ntile = BM // 128
    tiles_s = [scores[:, t * 128:(t + 1) * 128] for t in range(ntile)]
    m_old = m_s[...]
    m_new = m_old
    for t in range(ntile):
        m_new = jnp.maximum(m_new, tiles_s[t])
    s_acc = jnp.exp(tiles_s[0] - m_new)
    for t in range(1, ntile):
        s_acc = s_acc + jnp.exp(tiles_s[t] - m_new)
    s_s[...] = s_s[...] * jnp.exp(m_old - m_new) + s_acc
    improved = m_new > m_old
    idx = idx_s[...]
    for t in reversed(range(ntile)):
        idx = jnp.where(improved & (tiles_s[t] == m_new),
                        i * BM + t * 128 + lane_idx, idx)
    idx_s[...] = idx
    m_s[...] = m_new

    @pl.when(i == nblk - 1)
    def _finalize():
        m_lane = m_s[...]
        m_g = jnp.max(m_lane, axis=1, keepdims=True)
        s = jnp.sum(s_s[...] * jnp.exp(m_lane - m_g),
                    axis=1, keepdims=True)
        cand = jnp.where(m_lane == m_g, idx_s[...], jnp.int32(2 ** 30))
        slot = jnp.min(cand, axis=1, keepdims=True)
        best_w = 1.0 / s
        do_write = best_w > thr_ref[...]
        eq = slot == slot.reshape(1, B)
        ii = jax.lax.broadcasted_iota(jnp.int32, (B, B), 0)
        jj = jax.lax.broadcasted_iota(jnp.int32, (B, B), 1)
        conflict = jnp.any(eq & (jj > ii) & do_write.reshape(1, B),
                           axis=1, keepdims=True)
        final_write = do_write & jnp.logical_not(conflict)
        group = slot // G
        row = slot % G
        r8 = jax.lax.broadcasted_iota(jnp.int32, (1, G, 1), 1)
        match3 = (final_write.reshape(1, 1, B)
                  & (group.reshape(1, 1, B) == group.reshape(B, 1, 1))
                  & (row.reshape(1, 1, B) == r8))
        match2 = jnp.where(match3, 1.0, 0.0).reshape(B * G, B)
        mask_ref[...] = jnp.sum(match2, axis=1, keepdims=True)
        patch_ref[...] = jnp.dot(match2, x,
                                 preferred_element_type=jnp.float32)
        groups_ref[...] = group.reshape(1, B)


def _scatter_body(groups_ref, patch_ref, mask_ref, cur_ref, out_ref):
    out_ref[...] = jnp.where(mask_ref[...] != 0.0,
                             patch_ref[...], cur_ref[...])


@functools.partial(jax.jit, static_argnames=())
def kernel(input_data, memory, W1, b1, W2, b2, threshold):
    del b2
    M = memory.shape[0]
    nblk = M // BM

    w1a = W1[:F, :]
    w1bt = W1[F:, :].T
    b1r = b1.reshape(1, F)
    thr = threshold.reshape(1, 1)
    w2row = W2.reshape(1, F)
    w2blk = jnp.zeros((G, G * F), jnp.float32)
    for g in range(G):
        w2blk = w2blk.at[g:g + 1, g * F:(g + 1) * F].set(w2row)

    out_mem, patch, mask, groups = pl.pallas_call(
        _score_copy_body,
        grid=(nblk,),
        in_specs=[
            pl.BlockSpec((B, F), lambda i: (0, 0)),
            pl.BlockSpec((F, F), lambda i: (0, 0)),
            pl.BlockSpec((F, F), lambda i: (0, 0)),
            pl.BlockSpec((1, F), lambda i: (0, 0)),
            pl.BlockSpec((G, G * F), lambda i: (0, 0)),
            pl.BlockSpec((1, 1), lambda i: (0, 0)),
            pl.BlockSpec((BM, F), lambda i: (i, 0)),
        ],
        out_specs=[
            pl.BlockSpec((BM, F), lambda i: (i, 0)),
            pl.BlockSpec((B * G, F), lambda i: (0, 0)),
            pl.BlockSpec((B * G, 1), lambda i: (0, 0)),
            pl.BlockSpec((1, B), lambda i: (0, 0)),
        ],
        out_shape=[
            jax.ShapeDtypeStruct((M, F), jnp.float32),
            jax.ShapeDtypeStruct((B * G, F), jnp.float32),
            jax.ShapeDtypeStruct((B * G, 1), jnp.float32),
            jax.ShapeDtypeStruct((1, B), jnp.int32),
        ],
        scratch_shapes=[
            pltpu.VMEM((B, 128), jnp.float32),
            pltpu.VMEM((B, 128), jnp.float32),
            pltpu.VMEM((B, 128), jnp.int32),
        ],
    )(input_data, w1a, w1bt, b1r, w2blk, thr, memory)

    groups1d = groups.reshape(B)

    grid_spec = pltpu.PrefetchScalarGridSpec(
        num_scalar_prefetch=1,
        grid=(B,),
        in_specs=[
            pl.BlockSpec((G, F), lambda i, g: (i, 0)),
            pl.BlockSpec((G, 1), lambda i, g: (i, 0)),
            pl.BlockSpec((G, F), lambda i, g: (g[i], 0)),
        ],
        out_specs=pl.BlockSpec((G, F), lambda i, g: (g[i], 0)),
    )
    updated = pl.pallas_call(
        _scatter_body,
        grid_spec=grid_spec,
        out_shape=jax.ShapeDtypeStruct((M, F), jnp.float32),
        input_output_aliases={3: 0},
    )(groups1d, patch, mask, out_mem)
    return updated


# speedup vs baseline: 1.7408x; 1.0705x over previous
import functools

import jax
import jax.numpy as jnp
from jax.experimental import pallas as pl
from jax.experimental.pallas import tpu as pltpu

B = 32
F = 64
BM = 2048
G = 8


def _score_copy_body(x_ref, w1a_ref, w1bt_ref, b1_ref, w2blk_ref, thr_ref,
                     mem_ref, out_mem_ref, patch_ref, mask_ref, groups_ref,
                     m_s, s_s, idx_s):
    i = pl.program_id(0)
    nblk = pl.num_programs(0)

    @pl.when(i == 0)
    def _init():
        m_s[...] = jnp.full((B, 128), -jnp.inf, jnp.float32)
        s_s[...] = jnp.zeros((B, 128), jnp.float32)
        idx_s[...] = jnp.zeros((B, 128), jnp.int32)

    x = x_ref[...]
    in_proj = jnp.dot(x, w1a_ref[...],
                      preferred_element_type=jnp.float32) + b1_ref[...]
    memb = mem_ref[...]
    out_mem_ref[...] = memb
    mem_projT = jax.lax.dot_general(
        w1bt_ref[...], memb, (((1,), (1,)), ((), ())),
        preferred_element_type=jnp.float32)

    w2blk = w2blk_ref[...]
    tiles = []
    for g in range(B // G):
        ipg = in_proj[g * G:(g + 1) * G, :]
        slab = jnp.tanh(mem_projT[None, :, :]
                        + ipg[:, :, None]).reshape(G * F, BM)
        tiles.append(jnp.dot(w2blk, slab,
                             preferred_element_type=jnp.float32))
    scores = jnp.concatenate(tiles, axis=0)

    lane_idx = jax.lax.broadcasted_iota(jnp.int32, (B, 128), 1)
    ntile = BM // 128
    tiles_s = [scores[:, t * 128:(t + 1) * 128] for t in range(ntile)]
    m_old = m_s[...]
    m_new = m_old
    for t in range(ntile):
        m_new = jnp.maximum(m_new, tiles_s[t])
    s_acc = jnp.exp(tiles_s[0] - m_new)
    for t in range(1, ntile):
        s_acc = s_acc + jnp.exp(tiles_s[t] - m_new)
    s_s[...] = s_s[...] * jnp.exp(m_old - m_new) + s_acc
    improved = m_new > m_old
    idx = idx_s[...]
    for t in reversed(range(ntile)):
        idx = jnp.where(improved & (tiles_s[t] == m_new),
                        i * BM + t * 128 + lane_idx, idx)
    idx_s[...] = idx
    m_s[...] = m_new

    @pl.when(i == nblk - 1)
    def _finalize():
        m_lane = m_s[...]
        m_g = jnp.max(m_lane, axis=1, keepdims=True)
        s = jnp.sum(s_s[...] * jnp.exp(m_lane - m_g),
                    axis=1, keepdims=True)
        cand = jnp.where(m_lane == m_g, idx_s[...], jnp.int32(2 ** 30))
        slot = jnp.min(cand, axis=1, keepdims=True)
        best_w = 1.0 / s
        do_write = best_w > thr_ref[...]
        eq = slot == slot.reshape(1, B)
        ii = jax.lax.broadcasted_iota(jnp.int32, (B, B), 0)
        jj = jax.lax.broadcasted_iota(jnp.int32, (B, B), 1)
        conflict = jnp.any(eq & (jj > ii) & do_write.reshape(1, B),
                           axis=1, keepdims=True)
        final_write = do_write & jnp.logical_not(conflict)
        group = slot // G
        row = slot % G
        r8 = jax.lax.broadcasted_iota(jnp.int32, (1, G, 1), 1)
        match3 = (final_write.reshape(1, 1, B)
                  & (group.reshape(1, 1, B) == group.reshape(B, 1, 1))
                  & (row.reshape(1, 1, B) == r8))
        match2 = jnp.where(match3, 1.0, 0.0).reshape(B * G, B)
        mask_ref[...] = jnp.sum(match2, axis=1, keepdims=True)
        patch_ref[...] = jnp.dot(match2, x,
                                 preferred_element_type=jnp.float32)
        groups_ref[...] = group.reshape(1, B)


def _scatter_body(groups_ref, patch_ref, mask_ref, cur_ref, out_ref):
    out_ref[...] = jnp.where(mask_ref[...] != 0.0,
                             patch_ref[...], cur_ref[...])


@functools.partial(jax.jit, static_argnames=())
def kernel(input_data, memory, W1, b1, W2, b2, threshold):
    del b2
    M = memory.shape[0]
    nblk = M // BM

    w1a = W1[:F, :]
    w1bt = W1[F:, :].T
    b1r = b1.reshape(1, F)
    thr = threshold.reshape(1, 1)
    w2row = W2.reshape(1, F)
    w2blk = jnp.zeros((G, G * F), jnp.float32)
    for g in range(G):
        w2blk = w2blk.at[g:g + 1, g * F:(g + 1) * F].set(w2row)

    out_mem, patch, mask, groups = pl.pallas_call(
        _score_copy_body,
        grid=(nblk,),
        in_specs=[
            pl.BlockSpec((B, F), lambda i: (0, 0)),
            pl.BlockSpec((F, F), lambda i: (0, 0)),
            pl.BlockSpec((F, F), lambda i: (0, 0)),
            pl.BlockSpec((1, F), lambda i: (0, 0)),
            pl.BlockSpec((G, G * F), lambda i: (0, 0)),
            pl.BlockSpec((1, 1), lambda i: (0, 0)),
            pl.BlockSpec((BM, F), lambda i: (i, 0)),
        ],
        out_specs=[
            pl.BlockSpec((BM, F), lambda i: (i, 0)),
            pl.BlockSpec((B * G, F), lambda i: (0, 0)),
            pl.BlockSpec((B * G, 1), lambda i: (0, 0)),
            pl.BlockSpec((1, B), lambda i: (0, 0)),
        ],
        out_shape=[
            jax.ShapeDtypeStruct((M, F), jnp.float32),
            jax.ShapeDtypeStruct((B * G, F), jnp.float32),
            jax.ShapeDtypeStruct((B * G, 1), jnp.float32),
            jax.ShapeDtypeStruct((1, B), jnp.int32),
        ],
        scratch_shapes=[
            pltpu.VMEM((B, 128), jnp.float32),
            pltpu.VMEM((B, 128), jnp.float32),
            pltpu.VMEM((B, 128), jnp.int32),
        ],
    )(input_data, w1a, w1bt, b1r, w2blk, thr, memory)

    groups1d = groups.reshape(B)

    grid_spec = pltpu.PrefetchScalarGridSpec(
        num_scalar_prefetch=1,
        grid=(B,),
        in_specs=[
            pl.BlockSpec((G, F), lambda i, g: (i, 0)),
            pl.BlockSpec((G, 1), lambda i, g: (i, 0)),
            pl.BlockSpec((G, F), lambda i, g: (g[i], 0)),
        ],
        out_specs=pl.BlockSpec((G, F), lambda i, g: (g[i], 0)),
    )
    updated = pl.pallas_call(
        _scatter_body,
        grid_spec=grid_spec,
        out_shape=jax.ShapeDtypeStruct((M, F), jnp.float32),
        input_output_aliases={3: 0},
    )(groups1d, patch, mask, out_mem)
    return updated


# speedup vs baseline: 1.7991x; 1.0335x over previous
import functools

import jax
import jax.numpy as jnp
from jax.experimental import pallas as pl
from jax.experimental.pallas import tpu as pltpu

B = 32
F = 64
BM = 4096
G = 8


def _score_copy_body(x_ref, w1a_ref, w1bt_ref, b1_ref, w2blk_ref, thr_ref,
                     mem_ref, out_mem_ref, patch_ref, mask_ref, groups_ref,
                     m_s, s_s, idx_s):
    i = pl.program_id(0)
    nblk = pl.num_programs(0)

    @pl.when(i == 0)
    def _init():
        m_s[...] = jnp.full((B, 128), -jnp.inf, jnp.float32)
        s_s[...] = jnp.zeros((B, 128), jnp.float32)
        idx_s[...] = jnp.zeros((B, 128), jnp.int32)

    x = x_ref[...]
    in_proj = jnp.dot(x, w1a_ref[...],
                      preferred_element_type=jnp.float32) + b1_ref[...]
    memb = mem_ref[...]
    out_mem_ref[...] = memb
    mem_projT = jax.lax.dot_general(
        w1bt_ref[...], memb, (((1,), (1,)), ((), ())),
        preferred_element_type=jnp.float32)

    w2blk = w2blk_ref[...]
    tiles = []
    for g in range(B // G):
        ipg = in_proj[g * G:(g + 1) * G, :]
        slab = jnp.tanh(mem_projT[None, :, :]
                        + ipg[:, :, None]).reshape(G * F, BM)
        tiles.append(jnp.dot(w2blk, slab,
                             preferred_element_type=jnp.float32))
    scores = jnp.concatenate(tiles, axis=0)

    lane_idx = jax.lax.broadcasted_iota(jnp.int32, (B, 128), 1)
    ntile = BM // 128
    tiles_s = [scores[:, t * 128:(t + 1) * 128] for t in range(ntile)]
    m_old = m_s[...]
    m_new = m_old
    for t in range(ntile):
        m_new = jnp.maximum(m_new, tiles_s[t])
    s_acc = jnp.exp(tiles_s[0] - m_new)
    for t in range(1, ntile):
        s_acc = s_acc + jnp.exp(tiles_s[t] - m_new)
    s_s[...] = s_s[...] * jnp.exp(m_old - m_new) + s_acc
    improved = m_new > m_old
    idx = idx_s[...]
    for t in reversed(range(ntile)):
        idx = jnp.where(improved & (tiles_s[t] == m_new),
                        i * BM + t * 128 + lane_idx, idx)
    idx_s[...] = idx
    m_s[...] = m_new

    @pl.when(i == nblk - 1)
    def _finalize():
        m_lane = m_s[...]
        m_g = jnp.max(m_lane, axis=1, keepdims=True)
        s = jnp.sum(s_s[...] * jnp.exp(m_lane - m_g),
                    axis=1, keepdims=True)
        cand = jnp.where(m_lane == m_g, idx_s[...], jnp.int32(2 ** 30))
        slot = jnp.min(cand, axis=1, keepdims=True)
        best_w = 1.0 / s
        do_write = best_w > thr_ref[...]
        eq = slot == slot.reshape(1, B)
        ii = jax.lax.broadcasted_iota(jnp.int32, (B, B), 0)
        jj = jax.lax.broadcasted_iota(jnp.int32, (B, B), 1)
        conflict = jnp.any(eq & (jj > ii) & do_write.reshape(1, B),
                           axis=1, keepdims=True)
        final_write = do_write & jnp.logical_not(conflict)
        group = slot // G
        row = slot % G
        r8 = jax.lax.broadcasted_iota(jnp.int32, (1, G, 1), 1)
        match3 = (final_write.reshape(1, 1, B)
                  & (group.reshape(1, 1, B) == group.reshape(B, 1, 1))
                  & (row.reshape(1, 1, B) == r8))
        match2 = jnp.where(match3, 1.0, 0.0).reshape(B * G, B)
        mask_ref[...] = jnp.sum(match2, axis=1, keepdims=True)
        patch_ref[...] = jnp.dot(match2, x,
                                 preferred_element_type=jnp.float32)
        groups_ref[...] = group.reshape(1, B)


def _scatter_body(groups_ref, patch_ref, mask_ref, cur_ref, out_ref):
    out_ref[...] = jnp.where(mask_ref[...] != 0.0,
                             patch_ref[...], cur_ref[...])


@functools.partial(jax.jit, static_argnames=())
def kernel(input_data, memory, W1, b1, W2, b2, threshold):
    del b2
    M = memory.shape[0]
    nblk = M // BM

    w1a = W1[:F, :]
    w1bt = W1[F:, :].T
    b1r = b1.reshape(1, F)
    thr = threshold.reshape(1, 1)
    w2row = W2.reshape(1, F)
    w2blk = jnp.zeros((G, G * F), jnp.float32)
    for g in range(G):
        w2blk = w2blk.at[g:g + 1, g * F:(g + 1) * F].set(w2row)

    out_mem, patch, mask, groups = pl.pallas_call(
        _score_copy_body,
        grid=(nblk,),
        in_specs=[
            pl.BlockSpec((B, F), lambda i: (0, 0)),
            pl.BlockSpec((F, F), lambda i: (0, 0)),
            pl.BlockSpec((F, F), lambda i: (0, 0)),
            pl.BlockSpec((1, F), lambda i: (0, 0)),
            pl.BlockSpec((G, G * F), lambda i: (0, 0)),
            pl.BlockSpec((1, 1), lambda i: (0, 0)),
            pl.BlockSpec((BM, F), lambda i: (i, 0)),
        ],
        out_specs=[
            pl.BlockSpec((BM, F), lambda i: (i, 0)),
            pl.BlockSpec((B * G, F), lambda i: (0, 0)),
            pl.BlockSpec((B * G, 1), lambda i: (0, 0)),
            pl.BlockSpec((1, B), lambda i: (0, 0)),
        ],
        out_shape=[
            jax.ShapeDtypeStruct((M, F), jnp.float32),
            jax.ShapeDtypeStruct((B * G, F), jnp.float32),
            jax.ShapeDtypeStruct((B * G, 1), jnp.float32),
            jax.ShapeDtypeStruct((1, B), jnp.int32),
        ],
        scratch_shapes=[
            pltpu.VMEM((B, 128), jnp.float32),
            pltpu.VMEM((B, 128), jnp.float32),
            pltpu.VMEM((B, 128), jnp.int32),
        ],
    )(input_data, w1a, w1bt, b1r, w2blk, thr, memory)

    groups1d = groups.reshape(B)

    grid_spec = pltpu.PrefetchScalarGridSpec(
        num_scalar_prefetch=1,
        grid=(B,),
        in_specs=[
            pl.BlockSpec((G, F), lambda i, g: (i, 0)),
            pl.BlockSpec((G, 1), lambda i, g: (i, 0)),
            pl.BlockSpec((G, F), lambda i, g: (g[i], 0)),
        ],
        out_specs=pl.BlockSpec((G, F), lambda i, g: (g[i], 0)),
    )
    updated = pl.pallas_call(
        _scatter_body,
        grid_spec=grid_spec,
        out_shape=jax.ShapeDtypeStruct((M, F), jnp.float32),
        input_output_aliases={3: 0},
    )(groups1d, patch, mask, out_mem)
    return updated


# speedup vs baseline: 1.8008x; 1.0009x over previous
import functools

import jax
import jax.numpy as jnp
from jax.experimental import pallas as pl
from jax.experimental.pallas import tpu as pltpu

B = 32
F = 64
BM = 8192
G = 8


def _score_copy_body(x_ref, w1a_ref, w1bt_ref, b1_ref, w2blk_ref, thr_ref,
                     mem_ref, out_mem_ref, patch_ref, mask_ref, groups_ref,
                     m_s, s_s, idx_s):
    i = pl.program_id(0)
    nblk = pl.num_programs(0)

    @pl.when(i == 0)
    def _init():
        m_s[...] = jnp.full((B, 128), -jnp.inf, jnp.float32)
        s_s[...] = jnp.zeros((B, 128), jnp.float32)
        idx_s[...] = jnp.zeros((B, 128), jnp.int32)

    x = x_ref[...]
    in_proj = jnp.dot(x, w1a_ref[...],
                      preferred_element_type=jnp.float32) + b1_ref[...]
    memb = mem_ref[...]
    out_mem_ref[...] = memb
    mem_projT = jax.lax.dot_general(
        w1bt_ref[...], memb, (((1,), (1,)), ((), ())),
        preferred_element_type=jnp.float32)

    w2blk = w2blk_ref[...]
    tiles = []
    for g in range(B // G):
        ipg = in_proj[g * G:(g + 1) * G, :]
        slab = jnp.tanh(mem_projT[None, :, :]
                        + ipg[:, :, None]).reshape(G * F, BM)
        tiles.append(jnp.dot(w2blk, slab,
                             preferred_element_type=jnp.float32))
    scores = jnp.concatenate(tiles, axis=0)

    lane_idx = jax.lax.broadcasted_iota(jnp.int32, (B, 128), 1)
    ntile = BM // 128
    tiles_s = [scores[:, t * 128:(t + 1) * 128] for t in range(ntile)]
    m_old = m_s[...]
    m_new = m_old
    for t in range(ntile):
        m_new = jnp.maximum(m_new, tiles_s[t])
    s_acc = jnp.exp(tiles_s[0] - m_new)
    for t in range(1, ntile):
        s_acc = s_acc + jnp.exp(tiles_s[t] - m_new)
    s_s[...] = s_s[...] * jnp.exp(m_old - m_new) + s_acc
    improved = m_new > m_old
    idx = idx_s[...]
    for t in reversed(range(ntile)):
        idx = jnp.where(improved & (tiles_s[t] == m_new),
                        i * BM + t * 128 + lane_idx, idx)
    idx_s[...] = idx
    m_s[...] = m_new

    @pl.when(i == nblk - 1)
    def _finalize():
        m_lane = m_s[...]
        m_g = jnp.max(m_lane, axis=1, keepdims=True)
        s = jnp.sum(s_s[...] * jnp.exp(m_lane - m_g),
                    axis=1, keepdims=True)
        cand = jnp.where(m_lane == m_g, idx_s[...], jnp.int32(2 ** 30))
        slot = jnp.min(cand, axis=1, keepdims=True)
        best_w = 1.0 / s
        do_write = best_w > thr_ref[...]
        eq = slot == slot.reshape(1, B)
        ii = jax.lax.broadcasted_iota(jnp.int32, (B, B), 0)
        jj = jax.lax.broadcasted_iota(jnp.int32, (B, B), 1)
        conflict = jnp.any(eq & (jj > ii) & do_write.reshape(1, B),
                           axis=1, keepdims=True)
        final_write = do_write & jnp.logical_not(conflict)
        group = slot // G
        row = slot % G
        r8 = jax.lax.broadcasted_iota(jnp.int32, (1, G, 1), 1)
        match3 = (final_write.reshape(1, 1, B)
                  & (group.reshape(1, 1, B) == group.reshape(B, 1, 1))
                  & (row.reshape(1, 1, B) == r8))
        match2 = jnp.where(match3, 1.0, 0.0).reshape(B * G, B)
        mask_ref[...] = jnp.sum(match2, axis=1, keepdims=True)
        patch_ref[...] = jnp.dot(match2, x,
                                 preferred_element_type=jnp.float32)
        groups_ref[...] = group.reshape(1, B)


def _scatter_body(groups_ref, patch_ref, mask_ref, cur_ref, out_ref):
    out_ref[...] = jnp.where(mask_ref[...] != 0.0,
                             patch_ref[...], cur_ref[...])


@functools.partial(jax.jit, static_argnames=())
def kernel(input_data, memory, W1, b1, W2, b2, threshold):
    del b2
    M = memory.shape[0]
    nblk = M // BM

    w1a = W1[:F, :]
    w1bt = W1[F:, :].T
    b1r = b1.reshape(1, F)
    thr = threshold.reshape(1, 1)
    w2row = W2.reshape(1, F)
    w2blk = jnp.zeros((G, G * F), jnp.float32)
    for g in range(G):
        w2blk = w2blk.at[g:g + 1, g * F:(g + 1) * F].set(w2row)

    out_mem, patch, mask, groups = pl.pallas_call(
        _score_copy_body,
        grid=(nblk,),
        in_specs=[
            pl.BlockSpec((B, F), lambda i: (0, 0)),
            pl.BlockSpec((F, F), lambda i: (0, 0)),
            pl.BlockSpec((F, F), lambda i: (0, 0)),
            pl.BlockSpec((1, F), lambda i: (0, 0)),
            pl.BlockSpec((G, G * F), lambda i: (0, 0)),
            pl.BlockSpec((1, 1), lambda i: (0, 0)),
            pl.BlockSpec((BM, F), lambda i: (i, 0)),
        ],
        out_specs=[
            pl.BlockSpec((BM, F), lambda i: (i, 0)),
            pl.BlockSpec((B * G, F), lambda i: (0, 0)),
            pl.BlockSpec((B * G, 1), lambda i: (0, 0)),
            pl.BlockSpec((1, B), lambda i: (0, 0)),
        ],
        out_shape=[
            jax.ShapeDtypeStruct((M, F), jnp.float32),
            jax.ShapeDtypeStruct((B * G, F), jnp.float32),
            jax.ShapeDtypeStruct((B * G, 1), jnp.float32),
            jax.ShapeDtypeStruct((1, B), jnp.int32),
        ],
        scratch_shapes=[
            pltpu.VMEM((B, 128), jnp.float32),
            pltpu.VMEM((B, 128), jnp.float32),
            pltpu.VMEM((B, 128), jnp.int32),
        ],
    )(input_data, w1a, w1bt, b1r, w2blk, thr, memory)

    groups1d = groups.reshape(B)

    grid_spec = pltpu.PrefetchScalarGridSpec(
        num_scalar_prefetch=1,
        grid=(B,),
        in_specs=[
            pl.BlockSpec((G, F), lambda i, g: (i, 0)),
            pl.BlockSpec((G, 1), lambda i, g: (i, 0)),
            pl.BlockSpec((G, F), lambda i, g: (g[i], 0)),
        ],
        out_specs=pl.BlockSpec((G, F), lambda i, g: (g[i], 0)),
    )
    updated = pl.pallas_call(
        _scatter_body,
        grid_spec=grid_spec,
        out_shape=jax.ShapeDtypeStruct((M, F), jnp.float32),
        input_output_aliases={3: 0},
    )(groups1d, patch, mask, out_mem)
    return updated
